# baseline (speedup 1.0000x reference)
import jax
import jax.numpy as jnp
from jax.experimental import pallas as pl
from jax.experimental.pallas import tpu as pltpu


_NSLOT = 4


def _block_sizes(n):
    for bm in (200, 80, 16, 8):
        if n % bm == 0:
            for bmc in (40, 16, 8):
                if bm % bmc == 0:
                    return bm, bmc
    return n, n


def _chunk_copies(adj_hbm, buf_ref, sems, blk, slot, bm, bmc):
    n_chunks = bm // bmc
    return [
        pltpu.make_async_copy(
            adj_hbm.at[pl.ds(blk * bm + c * bmc, bmc), :],
            buf_ref.at[slot, pl.ds(c * bmc, bmc), :],
            sems.at[slot, c],
        )
        for c in range(n_chunks)
    ]


def _pipeline_block(adj_hbm, buf_ref, sems, nb, bm, bmc):
    i = pl.program_id(0)
    slot = jax.lax.rem(i, _NSLOT)

    @pl.when(i == 0)
    def _():
        for lead in range(_NSLOT - 1):
            if lead < nb:
                for cp in _chunk_copies(adj_hbm, buf_ref, sems, lead, lead, bm, bmc):
                    cp.start()

    nxt_blk = i + _NSLOT - 1

    @pl.when(nxt_blk < nb)
    def _():
        for cp in _chunk_copies(
            adj_hbm, buf_ref, sems, nxt_blk, jax.lax.rem(nxt_blk, _NSLOT), bm, bmc
        ):
            cp.start()

    for cp in _chunk_copies(adj_hbm, buf_ref, sems, i, slot, bm, bmc):
        cp.wait()
    return buf_ref[slot]


def _h_kernel(nb, bm, bmc, x_ref, w0_ref, adj_hbm, h_ref, xw0_ref, buf_ref, sems):
    @pl.when(pl.program_id(0) == 0)
    def _():
        xw0_ref[...] = jnp.dot(
            x_ref[...], w0_ref[...], preferred_element_type=jnp.float32
        ).astype(jnp.bfloat16)

    adj_blk = _pipeline_block(adj_hbm, buf_ref, sems, nb, bm, bmc)
    h_ref[...] = jnp.tanh(
        jnp.dot(
            adj_blk.astype(jnp.bfloat16),
            xw0_ref[...],
            preferred_element_type=jnp.float32,
        )
    )


def _head_kernel(
    nb, bm, bmc, h_ref, wcat_ref, adj_hbm, eps_ref,
    z_ref, m_ref, s_ref, hw_ref, buf_ref, sems,
):
    latent = m_ref.shape[1]

    @pl.when(pl.program_id(0) == 0)
    def _():
        hw_ref[...] = jnp.dot(
            h_ref[...], wcat_ref[...], preferred_element_type=jnp.float32
        ).astype(jnp.bfloat16)

    adj_blk = _pipeline_block(adj_hbm, buf_ref, sems, nb, bm, bmc)
    acc = jnp.dot(
        adj_blk.astype(jnp.bfloat16), hw_ref[...], preferred_element_type=jnp.float32
    )
    m = acc[:, :latent]
    s = jnp.maximum(acc[:, latent:], 0.0) + 0.0001
    m_ref[...] = m
    s_ref[...] = s
    z_ref[...] = eps_ref[...] * s + m


def kernel(adj, x, W0, Wm, Ws):
    import functools

    n, d_in = x.shape
    hidden = W0.shape[1]
    latent = Wm.shape[1]
    bm, bmc = _block_sizes(n)
    nb = n // bm
    n_chunks = bm // bmc
    grid = (nb,)

    adj_spec = pl.BlockSpec(memory_space=pl.ANY)
    dma_scratch = [
        pltpu.VMEM((_NSLOT, bm, n), jnp.float32),
        pltpu.SemaphoreType.DMA((_NSLOT, n_chunks)),
    ]

    h = pl.pallas_call(
        functools.partial(_h_kernel, nb, bm, bmc),
        grid=grid,
        in_specs=[
            pl.BlockSpec((n, d_in), lambda i: (0, 0)),
            pl.BlockSpec((d_in, hidden), lambda i: (0, 0)),
            adj_spec,
        ],
        out_specs=pl.BlockSpec((bm, hidden), lambda i: (i, 0)),
        out_shape=jax.ShapeDtypeStruct((n, hidden), jnp.float32),
        scratch_shapes=[pltpu.VMEM((n, hidden), jnp.bfloat16)] + dma_scratch,
        compiler_params=pltpu.CompilerParams(
            dimension_semantics=("arbitrary",),
        ),
    )(x, W0, adj)

    wcat = jnp.concatenate([Wm, Ws], axis=1)
    eps = jax.random.normal(jax.random.key(42), (n, latent), dtype=jnp.float32)

    out_sds = jax.ShapeDtypeStruct((n, latent), jnp.float32)
    lat_spec = pl.BlockSpec((bm, latent), lambda i: (i, 0))
    z, m_q_z, std_q_z = pl.pallas_call(
        functools.partial(_head_kernel, nb, bm, bmc),
        grid=grid,
        in_specs=[
            pl.BlockSpec((n, hidden), lambda i: (0, 0)),
            pl.BlockSpec((hidden, 2 * latent), lambda i: (0, 0)),
            adj_spec,
            lat_spec,
        ],
        out_specs=[lat_spec, lat_spec, lat_spec],
        out_shape=[out_sds, out_sds, out_sds],
        scratch_shapes=[pltpu.VMEM((n, 2 * latent), jnp.bfloat16)] + dma_scratch,
        compiler_params=pltpu.CompilerParams(
            dimension_semantics=("arbitrary",),
        ),
    )(h, wcat, adj, eps)

    return (z, m_q_z, std_q_z)
